# baseline (speedup 1.0000x reference)
import functools

import jax
import jax.numpy as jnp
from jax import lax
from jax.experimental import pallas as pl
from jax.experimental.pallas import tpu as pltpu
from jax.experimental.pallas import tpu_sc as plsc

DIM = 128
C = 32
CP1 = C + 1

_INTERPRET = False

_NC = 2
_NS = 16
_NW = _NC * _NS
_CH = 128


_NB = 4


def _sc_stage(idx_hbm, tab, out_hbm, span, out_base, pair, wid,
              idx_v, rows_v, sum_v, gs, ws):
    base = wid * span
    nch = span // _CH
    half = _CH // 2
    obase = out_base + (base // 2 if pair else base)

    boff = pl.multiple_of(base, _CH)
    pltpu.sync_copy(idx_hbm.at[pl.ds(boff, span)], idx_v.at[pl.ds(0, span)])

    def fire_gather(c, b):
        off = pl.multiple_of(c * _CH, _CH)
        pltpu.async_copy(tab.at[idx_v.at[pl.ds(off, _CH)]], rows_v[b], gs[b])

    def wait_gather(b):
        pltpu.make_async_copy(
            tab.at[idx_v.at[pl.ds(0, _CH)]], rows_v[b], gs[b]).wait()

    def psum(b):
        def step(k, carry):
            for d in range(DIM // 16):
                sl = pl.ds(d * 16, 16)
                sum_v[b][k, sl] = (rows_v[b][2 * k, sl]
                                   + rows_v[b][2 * k + 1, sl])
            return carry

        lax.fori_loop(0, half, step, 0, unroll=4)

    def fire_write(c, b):
        if pair:
            ooff = pl.multiple_of(obase + c * half, half)
            pltpu.async_copy(sum_v[b], out_hbm.at[pl.ds(ooff, half)], ws[b])
        else:
            ooff = pl.multiple_of(obase + c * _CH, _CH)
            pltpu.async_copy(rows_v[b], out_hbm.at[pl.ds(ooff, _CH)], ws[b])

    def wait_write(b):
        src = sum_v[b] if pair else rows_v[b]
        n = half if pair else _CH
        pltpu.make_async_copy(src, out_hbm.at[pl.ds(0, n)], ws[b]).wait()

    if nch < 2 * _NB:
        def sbody(c, carry):
            fire_gather(c, 0)
            wait_gather(0)
            if pair:
                psum(0)
            fire_write(c, 0)
            wait_write(0)
            return carry

        lax.fori_loop(0, nch, sbody, 0)
        return

    fire_gather(0, 0)
    fire_gather(1, 1)

    def body(i, carry):
        for b in range(_NB):
            c = _NB * i + b
            b2 = (b + 2) % _NB

            @pl.when(jnp.logical_and(c >= 2, c + 2 < nch))
            def _():
                wait_write(b2)

            @pl.when(c + 2 < nch)
            def _():
                fire_gather(c + 2, b2)

            wait_gather(b)
            if pair:
                psum(b)
            fire_write(c, b)
        return carry

    lax.fori_loop(0, nch // _NB, body, 0)
    for b in range(_NB):
        wait_write(b)


def _sc_gather_body(ectx_idx, eemb_idx, remb_idx,
                    etab, eemb, remb,
                    adj_out, o_out, *scratch):
    wid = lax.axis_index("s") * _NC + lax.axis_index("c")
    idx_v = scratch[0]
    rows_v = scratch[1:1 + _NB]
    sum_v = None
    gs = scratch[1 + _NB:1 + 2 * _NB]
    ws = scratch[1 + 2 * _NB:1 + 3 * _NB]
    n_e = ectx_idx.shape[0]
    n_oe = eemb_idx.shape[0]
    n_or = remb_idx.shape[0]
    _sc_stage(ectx_idx, etab, adj_out, n_e // _NW, 0, False, wid,
              idx_v, rows_v, sum_v, gs, ws)
    _sc_stage(eemb_idx, eemb, o_out, n_oe // _NW, 0, False, wid,
              idx_v, rows_v, sum_v, gs, ws)
    _sc_stage(remb_idx, remb, o_out, n_or // _NW, n_oe, False, wid,
              idx_v, rows_v, sum_v, gs, ws)


def _sc_gather_half(ectx_idx, eemb_idx, remb_idx, etab, eemb, remb):
    mesh = plsc.VectorSubcoreMesh(core_axis_name="c", subcore_axis_name="s")
    n_adj = ectx_idx.shape[0]
    n_o = eemb_idx.shape[0] + remb_idx.shape[0]
    f = pl.kernel(
        _sc_gather_body,
        out_type=[
            jax.ShapeDtypeStruct((n_adj, DIM), jnp.float32),
            jax.ShapeDtypeStruct((n_o, DIM), jnp.float32),
        ],
        mesh=mesh,
        scratch_types=(
            [pltpu.VMEM((ectx_idx.shape[0] // _NW,), jnp.int32)]
            + [pltpu.VMEM((_CH, DIM), jnp.float32)] * _NB
            + [pltpu.SemaphoreType.DMA] * (2 * _NB)
        ),
    )
    return f(ectx_idx, eemb_idx, remb_idx, etab, eemb, remb)


def _branch_core(o, adj_w, a3, w, v, gate, bb):
    o_w = jnp.dot(o, w, preferred_element_type=jnp.float32)
    a3r = jnp.concatenate([a3[..., 1:], a3[..., :1]], axis=-1)
    vw3 = jnp.concatenate([adj_w.reshape(bb, C, DIM), o_w[:, None, :]],
                          axis=1)
    s3 = lax.dot_general(a3r, vw3, (((2,), (1,)), ((0,), (0,))),
                         preferred_element_type=jnp.float32)
    h = jnp.maximum(s3, 0.0)
    tmp = jnp.maximum(h * o[:, None, :], 0.0)
    score3 = jnp.sum(tmp * v[None, None, :], axis=2,
                     keepdims=True)
    m3 = jnp.max(score3, axis=1, keepdims=True)
    e3 = jnp.exp(score3 - m3)
    alpha3 = e3 / jnp.sum(e3, axis=1, keepdims=True)
    sg = jnp.sum(alpha3 * h, axis=1)
    g = jax.nn.sigmoid(gate)
    return g[None, :] * o + (1.0 - g[None, :]) * sg


_RT = 512


def _fused_body(o_ref, adj_ref, a_ref, i1_ref, i2_ref, rtab_ref,
                w_ref, v_ref, gate_ref, d_ref, acc, *, bb):
    br = pl.program_id(1)
    w = w_ref[0]
    o = o_ref[0]

    @pl.when(br < 2)
    def _():
        adj_w = jnp.dot(adj_ref[0].reshape(bb * C, DIM), w,
                        preferred_element_type=jnp.float32)
        out = _branch_core(o, adj_w, a_ref[0], w, v_ref[0, 0],
                           gate_ref[0, 0], bb)

        @pl.when(br == 0)
        def _():
            acc[...] = out

        @pl.when(br == 1)
        def _():
            acc[...] = acc[...] - out

    @pl.when(br == 2)
    def _():
        iota = lax.broadcasted_iota(jnp.int32, (bb * C, _RT), 1)
        oh = ((iota == i1_ref[...]).astype(jnp.float32)
              + (iota == i2_ref[...]).astype(jnp.float32))
        rtw = jnp.dot(rtab_ref[...], w, preferred_element_type=jnp.float32)
        adj_w = jnp.dot(oh, rtw, preferred_element_type=jnp.float32)
        out = _branch_core(o, adj_w, a_ref[0], w, v_ref[0, 0],
                           gate_ref[0, 0], bb)
        d_ref[0] = acc[...] + out


def _fused(o_all, adj_all, a_stack, i1, i2, rtab, w_pair, v_pair, gate_pair,
           *, bb):
    batch = o_all.shape[1]
    grid = (batch // bb, 3)
    return pl.pallas_call(
        functools.partial(_fused_body, bb=bb),
        grid=grid,
        in_specs=[
            pl.BlockSpec((1, bb, DIM), lambda g, b: (b, g, 0)),
            pl.BlockSpec((1, bb, C, DIM),
                         lambda g, b: (b - b // 2, g, 0, 0)),
            pl.BlockSpec((1, bb, CP1, CP1), lambda g, b: (b, g, 0, 0)),
            pl.BlockSpec((bb * C, 1), lambda g, b: (g, 0)),
            pl.BlockSpec((bb * C, 1), lambda g, b: (g, 0)),
            pl.BlockSpec((_RT, DIM), lambda g, b: (0, 0)),
            pl.BlockSpec((1, DIM, DIM), lambda g, b: (b // 2, 0, 0)),
            pl.BlockSpec((1, 1, DIM), lambda g, b: (b // 2, 0, 0)),
            pl.BlockSpec((1, 1, DIM), lambda g, b: (b // 2, 0, 0)),
        ],
        out_specs=[
            pl.BlockSpec((1, bb, DIM), lambda g, b: (0, g, 0)),
        ],
        out_shape=[
            jax.ShapeDtypeStruct((1, batch, DIM), jnp.float32),
        ],
        scratch_shapes=[
            pltpu.VMEM((bb, DIM), jnp.float32),
        ],
        interpret=_INTERPRET,
    )(o_all, adj_all, a_stack, i1, i2, rtab, w_pair, v_pair, gate_pair)[0]


def _score_body(dp_ref, dn_ref, p_ref, n_ref):
    p_ref[...] = jnp.sum(jnp.abs(dp_ref[0]), axis=1)
    n_ref[...] = jnp.sum(jnp.abs(dn_ref[0]), axis=1)


def _scores(dp, dn, *, bs):
    batch = dp.shape[1]
    grid = (batch // bs,)
    return pl.pallas_call(
        _score_body,
        grid=grid,
        in_specs=[
            pl.BlockSpec((1, bs, DIM), lambda g: (0, g, 0)),
            pl.BlockSpec((1, bs, DIM), lambda g: (0, g, 0)),
        ],
        out_specs=[
            pl.BlockSpec((bs,), lambda g: (g,)),
            pl.BlockSpec((bs,), lambda g: (g,)),
        ],
        out_shape=[
            jax.ShapeDtypeStruct((batch,), jnp.float32),
            jax.ShapeDtypeStruct((batch,), jnp.float32),
        ],
        interpret=_INTERPRET,
    )(dp, dn)


def kernel(epoch, pos_h, pos_r, pos_t, neg_h, neg_r, neg_t,
           ph_A, pr_A, pt_A, nh_A, nr_A, nt_A,
           ph_ctx, pt_ctx, nh_ctx, nt_ctx, pr_ctx, nr_ctx,
           entity_emb, relation_emb, entity_context, relation_context,
           entity_gcn_weight, relation_gcn_weight,
           gate_entity, gate_relation, v_ent, v_rel):
    batch = pos_h.shape[0]

    i32 = jnp.int32
    w_pair = jnp.stack([entity_gcn_weight, relation_gcn_weight])
    v_pair = jnp.stack([v_ent, v_rel]).reshape(2, 1, DIM)
    gate_pair = jnp.stack([gate_entity, gate_relation]).reshape(2, 1, DIM)
    bb = min(64, batch)

    rtab = jnp.zeros((_RT, DIM), jnp.float32).at[
        :relation_context.shape[0]].set(relation_context)

    def half(h_ctx, t_ctx, r_ctx, h_idx, t_idx, r_idx, h_A, t_A, r_A):
        ectx_idx = jnp.concatenate(
            [h_ctx, t_ctx], axis=0).astype(i32).reshape(-1)
        eemb_idx = jnp.concatenate([h_idx, t_idx], axis=0).astype(i32)
        remb_idx = r_idx.astype(i32)
        adj_rows, o_rows = _sc_gather_half(
            ectx_idx, eemb_idx, remb_idx,
            entity_context, entity_emb, relation_emb)
        adj_all = adj_rows.reshape(2, batch, C, DIM)
        o_all = o_rows.reshape(3, batch, DIM)
        rc = r_ctx.astype(i32).reshape(batch * C, 2)
        i1 = rc[:, 0:1]
        i2 = rc[:, 1:2]
        a_stack = jnp.stack([h_A, t_A, r_A])
        return _fused(o_all, adj_all, a_stack, i1, i2, rtab,
                      w_pair, v_pair, gate_pair, bb=bb)

    dp = half(ph_ctx, pt_ctx, pr_ctx, pos_h, pos_t, pos_r, ph_A, pt_A, pr_A)
    dn = half(nh_ctx, nt_ctx, nr_ctx, neg_h, neg_t, neg_r, nh_A, nt_A, nr_A)
    p_score, n_score = _scores(dp, dn, bs=min(1024, batch))
    return p_score, n_score


# speedup vs baseline: 1.1447x; 1.1447x over previous
import functools

import jax
import jax.numpy as jnp
from jax import lax
from jax.experimental import pallas as pl
from jax.experimental.pallas import tpu as pltpu
from jax.experimental.pallas import tpu_sc as plsc

DIM = 128
C = 32
CP1 = C + 1

_INTERPRET = False

_NC = 2
_NS = 16
_NW = _NC * _NS
_CH = 128


_NB = 4


def _sc_stage(idx_hbm, tab, out_hbm, span, out_base, pair, wid,
              idx_v, rows_v, sum_v, gs, ws):
    base = wid * span
    nch = span // _CH
    half = _CH // 2
    obase = out_base + (base // 2 if pair else base)

    boff = pl.multiple_of(base, _CH)
    pltpu.sync_copy(idx_hbm.at[pl.ds(boff, span)], idx_v.at[pl.ds(0, span)])

    def fire_gather(c, b):
        off = pl.multiple_of(c * _CH, _CH)
        pltpu.async_copy(tab.at[idx_v.at[pl.ds(off, _CH)]], rows_v[b], gs[b])

    def wait_gather(b):
        pltpu.make_async_copy(
            tab.at[idx_v.at[pl.ds(0, _CH)]], rows_v[b], gs[b]).wait()

    def psum(b):
        def step(k, carry):
            for d in range(DIM // 16):
                sl = pl.ds(d * 16, 16)
                sum_v[b][k, sl] = (rows_v[b][2 * k, sl]
                                   + rows_v[b][2 * k + 1, sl])
            return carry

        lax.fori_loop(0, half, step, 0, unroll=4)

    def fire_write(c, b):
        if pair:
            ooff = pl.multiple_of(obase + c * half, half)
            pltpu.async_copy(sum_v[b], out_hbm.at[pl.ds(ooff, half)], ws[b])
        else:
            ooff = pl.multiple_of(obase + c * _CH, _CH)
            pltpu.async_copy(rows_v[b], out_hbm.at[pl.ds(ooff, _CH)], ws[b])

    def wait_write(b):
        src = sum_v[b] if pair else rows_v[b]
        n = half if pair else _CH
        pltpu.make_async_copy(src, out_hbm.at[pl.ds(0, n)], ws[b]).wait()

    if nch < 2 * _NB:
        def sbody(c, carry):
            fire_gather(c, 0)
            wait_gather(0)
            if pair:
                psum(0)
            fire_write(c, 0)
            wait_write(0)
            return carry

        lax.fori_loop(0, nch, sbody, 0)
        return

    fire_gather(0, 0)
    fire_gather(1, 1)

    def body(i, carry):
        for b in range(_NB):
            c = _NB * i + b
            b2 = (b + 2) % _NB

            @pl.when(jnp.logical_and(c >= 2, c + 2 < nch))
            def _():
                wait_write(b2)

            @pl.when(c + 2 < nch)
            def _():
                fire_gather(c + 2, b2)

            wait_gather(b)
            if pair:
                psum(b)
            fire_write(c, b)
        return carry

    lax.fori_loop(0, nch // _NB, body, 0)
    for b in range(_NB):
        wait_write(b)


def _sc_gather_body(ectx_idx, rctx_idx, eemb_idx, remb_idx,
                    etab, rtab, eemb, remb,
                    adj_out, o_out, *scratch):
    wid = lax.axis_index("s") * _NC + lax.axis_index("c")
    idx_v = scratch[0]
    rows_v = scratch[1:1 + _NB]
    sum_v = scratch[1 + _NB:1 + 2 * _NB]
    gs = scratch[1 + 2 * _NB:1 + 3 * _NB]
    ws = scratch[1 + 3 * _NB:1 + 4 * _NB]
    n_e = ectx_idx.shape[0]
    n_r = rctx_idx.shape[0]
    n_oe = eemb_idx.shape[0]
    n_or = remb_idx.shape[0]
    _sc_stage(ectx_idx, etab, adj_out, n_e // _NW, 0, False, wid,
              idx_v, rows_v, sum_v, gs, ws)
    _sc_stage(rctx_idx, rtab, adj_out, n_r // _NW, n_e, True, wid,
              idx_v, rows_v, sum_v, gs, ws)
    _sc_stage(eemb_idx, eemb, o_out, n_oe // _NW, 0, False, wid,
              idx_v, rows_v, sum_v, gs, ws)
    _sc_stage(remb_idx, remb, o_out, n_or // _NW, n_oe, False, wid,
              idx_v, rows_v, sum_v, gs, ws)


def _sc_gather_half(ectx_idx, rctx_idx, eemb_idx, remb_idx,
                    etab, rtab, eemb, remb):
    mesh = plsc.VectorSubcoreMesh(core_axis_name="c", subcore_axis_name="s")
    n_adj = ectx_idx.shape[0] + rctx_idx.shape[0] // 2
    n_o = eemb_idx.shape[0] + remb_idx.shape[0]
    f = pl.kernel(
        _sc_gather_body,
        out_type=[
            jax.ShapeDtypeStruct((n_adj, DIM), jnp.float32),
            jax.ShapeDtypeStruct((n_o, DIM), jnp.float32),
        ],
        mesh=mesh,
        scratch_types=(
            [pltpu.VMEM((max(ectx_idx.shape[0], rctx_idx.shape[0]) // _NW,),
                        jnp.int32)]
            + [pltpu.VMEM((_CH, DIM), jnp.float32)] * _NB
            + [pltpu.VMEM((_CH // 2, DIM), jnp.float32)] * _NB
            + [pltpu.SemaphoreType.DMA] * (2 * _NB)
        ),
    )
    return f(ectx_idx, rctx_idx, eemb_idx, remb_idx, etab, rtab, eemb, remb)


def _branch_core(o, adj_w, a3, w, v, gate, bb):
    o_w = jnp.dot(o, w, preferred_element_type=jnp.float32)
    a3r = jnp.concatenate([a3[..., 1:], a3[..., :1]], axis=-1)
    vw3 = jnp.concatenate([adj_w.reshape(bb, C, DIM), o_w[:, None, :]],
                          axis=1)
    s3 = lax.dot_general(a3r, vw3, (((2,), (1,)), ((0,), (0,))),
                         preferred_element_type=jnp.float32)
    h = jnp.maximum(s3, 0.0)
    tmp = jnp.maximum(h * o[:, None, :], 0.0)
    score3 = jnp.sum(tmp * v[None, None, :], axis=2,
                     keepdims=True)
    m3 = jnp.max(score3, axis=1, keepdims=True)
    e3 = jnp.exp(score3 - m3)
    alpha3 = e3 / jnp.sum(e3, axis=1, keepdims=True)
    sg = jnp.sum(alpha3 * h, axis=1)
    g = jax.nn.sigmoid(gate)
    return g[None, :] * o + (1.0 - g[None, :]) * sg


def _fused_body(o_ref, adj_ref, a_ref, w_ref, v_ref, gate_ref,
                d_ref, acc, *, bb):
    br = pl.program_id(1)
    w = w_ref[0]
    adj_w = jnp.dot(adj_ref[0].reshape(bb * C, DIM), w,
                    preferred_element_type=jnp.float32)
    out = _branch_core(o_ref[0], adj_w, a_ref[0], w, v_ref[0, 0],
                       gate_ref[0, 0], bb)

    @pl.when(br == 0)
    def _():
        acc[...] = out

    @pl.when(br == 1)
    def _():
        acc[...] = acc[...] - out

    @pl.when(br == 2)
    def _():
        d_ref[0] = acc[...] + out


def _fused(o_all, adj_all, a_stack, w_pair, v_pair, gate_pair, *, bb):
    batch = o_all.shape[1]
    grid = (batch // bb, 3)
    return pl.pallas_call(
        functools.partial(_fused_body, bb=bb),
        grid=grid,
        in_specs=[
            pl.BlockSpec((1, bb, DIM), lambda g, b: (b, g, 0)),
            pl.BlockSpec((1, bb, C, DIM), lambda g, b: (b, g, 0, 0)),
            pl.BlockSpec((1, bb, CP1, CP1), lambda g, b: (b, g, 0, 0)),
            pl.BlockSpec((1, DIM, DIM), lambda g, b: (b // 2, 0, 0)),
            pl.BlockSpec((1, 1, DIM), lambda g, b: (b // 2, 0, 0)),
            pl.BlockSpec((1, 1, DIM), lambda g, b: (b // 2, 0, 0)),
        ],
        out_specs=[
            pl.BlockSpec((1, bb, DIM), lambda g, b: (0, g, 0)),
        ],
        out_shape=[
            jax.ShapeDtypeStruct((1, batch, DIM), jnp.float32),
        ],
        scratch_shapes=[
            pltpu.VMEM((bb, DIM), jnp.float32),
        ],
        interpret=_INTERPRET,
    )(o_all, adj_all, a_stack, w_pair, v_pair, gate_pair)[0]


def _score_body(dp_ref, dn_ref, p_ref, n_ref):
    p_ref[...] = jnp.sum(jnp.abs(dp_ref[0]), axis=1)
    n_ref[...] = jnp.sum(jnp.abs(dn_ref[0]), axis=1)


def _scores(dp, dn, *, bs):
    batch = dp.shape[1]
    grid = (batch // bs,)
    return pl.pallas_call(
        _score_body,
        grid=grid,
        in_specs=[
            pl.BlockSpec((1, bs, DIM), lambda g: (0, g, 0)),
            pl.BlockSpec((1, bs, DIM), lambda g: (0, g, 0)),
        ],
        out_specs=[
            pl.BlockSpec((bs,), lambda g: (g,)),
            pl.BlockSpec((bs,), lambda g: (g,)),
        ],
        out_shape=[
            jax.ShapeDtypeStruct((batch,), jnp.float32),
            jax.ShapeDtypeStruct((batch,), jnp.float32),
        ],
        interpret=_INTERPRET,
    )(dp, dn)


def kernel(epoch, pos_h, pos_r, pos_t, neg_h, neg_r, neg_t,
           ph_A, pr_A, pt_A, nh_A, nr_A, nt_A,
           ph_ctx, pt_ctx, nh_ctx, nt_ctx, pr_ctx, nr_ctx,
           entity_emb, relation_emb, entity_context, relation_context,
           entity_gcn_weight, relation_gcn_weight,
           gate_entity, gate_relation, v_ent, v_rel):
    batch = pos_h.shape[0]

    i32 = jnp.int32
    w_pair = jnp.stack([entity_gcn_weight, relation_gcn_weight])
    v_pair = jnp.stack([v_ent, v_rel]).reshape(2, 1, DIM)
    gate_pair = jnp.stack([gate_entity, gate_relation]).reshape(2, 1, DIM)
    bb = min(64, batch)

    def half(h_ctx, t_ctx, r_ctx, h_idx, t_idx, r_idx, h_A, t_A, r_A):
        ectx_idx = jnp.concatenate(
            [h_ctx, t_ctx], axis=0).astype(i32).reshape(-1)
        rctx_idx = r_ctx.astype(i32).reshape(-1)
        eemb_idx = jnp.concatenate([h_idx, t_idx], axis=0).astype(i32)
        remb_idx = r_idx.astype(i32)
        adj_rows, o_rows = _sc_gather_half(
            ectx_idx, rctx_idx, eemb_idx, remb_idx,
            entity_context, relation_context, entity_emb, relation_emb)
        adj_all = adj_rows.reshape(3, batch, C, DIM)
        o_all = o_rows.reshape(3, batch, DIM)
        a_stack = jnp.stack([h_A, t_A, r_A])
        return _fused(o_all, adj_all, a_stack,
                      w_pair, v_pair, gate_pair, bb=bb)

    dp = half(ph_ctx, pt_ctx, pr_ctx, pos_h, pos_t, pos_r, ph_A, pt_A, pr_A)
    dn = half(nh_ctx, nt_ctx, nr_ctx, neg_h, neg_t, neg_r, nh_A, nt_A, nr_A)
    p_score, n_score = _scores(dp, dn, bs=min(1024, batch))
    return p_score, n_score
